# baseline (speedup 1.0000x reference)
import functools

import jax
import jax.numpy as jnp
from jax import lax
from jax.experimental import pallas as pl
from jax.experimental.pallas import tpu as pltpu
from jax.experimental.pallas import tpu_sc as plsc

BATCH = 16384
EMBD_DIM = 128
NC = 2
NS = 16
L = 16
NW = NC * NS
B_PER_W = BATCH // NW
CHUNK = 128
N_CHUNKS = B_PER_W // CHUNK


def _sc_body(user_hbm, item_hbm, wu_hbm, wi_hbm, out_hbm,
             idx_u, idx_v, rows_u, rows_v, outb, sem0, sem1):
    wid = lax.axis_index("s") * NC + lax.axis_index("c")
    base = wid * B_PER_W
    last_lane = lax.iota(jnp.int32, L) == (L - 1)

    ci_u = pltpu.async_copy(user_hbm.at[pl.ds(base, B_PER_W)], idx_u, sem0)
    ci_v = pltpu.async_copy(item_hbm.at[pl.ds(base, B_PER_W)], idx_v, sem0)
    ci_u.wait()
    ci_v.wait()

    sems = [sem0, sem1]

    def start_gathers(c):
        slot = c % 2
        cu = pltpu.async_copy(
            wu_hbm.at[idx_u.at[pl.ds(c * CHUNK, CHUNK)]],
            rows_u.at[slot], sems[slot])
        cv = pltpu.async_copy(
            wi_hbm.at[idx_v.at[pl.ds(c * CHUNK, CHUNK)]],
            rows_v.at[slot], sems[slot])
        return cu, cv

    pending = start_gathers(0)
    for c in range(N_CHUNKS):
        cu, cv = pending
        if c + 1 < N_CHUNKS:
            pending = start_gathers(c + 1)
        slot = c % 2

        cu.wait()
        cv.wait()

        @plsc.parallel_loop(0, CHUNK, unroll=4)
        def row_body(r):
            acc = rows_u[slot, r, pl.ds(0, L)] * rows_v[slot, r, pl.ds(0, L)]
            for k in range(1, EMBD_DIM // L):
                acc = acc + (rows_u[slot, r, pl.ds(k * L, L)] *
                             rows_v[slot, r, pl.ds(k * L, L)])
            total = plsc.cumsum(acc)
            plsc.store_scatter(outb,
                               [jnp.full((L,), c * CHUNK + r, jnp.int32)],
                               total, mask=last_lane)

    pltpu.sync_copy(outb, out_hbm.at[pl.ds(base, B_PER_W)])


@jax.jit
def _ratings(user, item, w_user, w_item):
    mesh = plsc.VectorSubcoreMesh(core_axis_name="c", subcore_axis_name="s")
    return pl.kernel(
        _sc_body,
        out_type=jax.ShapeDtypeStruct((BATCH,), jnp.float32),
        mesh=mesh,
        compiler_params=pltpu.CompilerParams(needs_layout_passes=False),
        scratch_types=[
            pltpu.VMEM((B_PER_W,), jnp.int32),
            pltpu.VMEM((B_PER_W,), jnp.int32),
            pltpu.VMEM((2, CHUNK, EMBD_DIM), jnp.float32),
            pltpu.VMEM((2, CHUNK, EMBD_DIM), jnp.float32),
            pltpu.VMEM((B_PER_W,), jnp.float32),
            pltpu.SemaphoreType.DMA,
            pltpu.SemaphoreType.DMA,
        ],
    )(user, item, w_user, w_item)


def kernel(user, item, W_user, W_item):
    return _ratings(user, item, W_user, W_item)


# speedup vs baseline: 1.0129x; 1.0129x over previous
import functools

import jax
import jax.numpy as jnp
from jax import lax
from jax.experimental import pallas as pl
from jax.experimental.pallas import tpu as pltpu
from jax.experimental.pallas import tpu_sc as plsc

BATCH = 16384
EMBD_DIM = 128
NC = 2
NS = 16
L = 16
NW = NC * NS
B_PER_W = BATCH // NW
CHUNK = 128
N_CHUNKS = B_PER_W // CHUNK
NSLOT = 3


def _sc_body(user_hbm, item_hbm, wu_hbm, wi_hbm, out_hbm,
             idx_u, idx_v, rows_u, rows_v, outb, sem0, sem1, sem2):
    wid = lax.axis_index("s") * NC + lax.axis_index("c")
    base = wid * B_PER_W
    last_lane = lax.iota(jnp.int32, L) == (L - 1)

    ci_u = pltpu.async_copy(user_hbm.at[pl.ds(base, B_PER_W)], idx_u, sem0)
    ci_v = pltpu.async_copy(item_hbm.at[pl.ds(base, B_PER_W)], idx_v, sem0)
    ci_u.wait()
    ci_v.wait()

    sems = [sem0, sem1, sem2]

    def start_gathers(c):
        slot = c % NSLOT
        cu = pltpu.async_copy(
            wu_hbm.at[idx_u.at[pl.ds(c * CHUNK, CHUNK)]],
            rows_u.at[slot], sems[slot])
        cv = pltpu.async_copy(
            wi_hbm.at[idx_v.at[pl.ds(c * CHUNK, CHUNK)]],
            rows_v.at[slot], sems[slot])
        return cu, cv

    pend = {0: start_gathers(0), 1: start_gathers(1)}
    for c in range(N_CHUNKS):
        cu, cv = pend[c % NSLOT]
        if c + 2 < N_CHUNKS:
            pend[(c + 2) % NSLOT] = start_gathers(c + 2)
        slot = c % NSLOT

        cu.wait()
        cv.wait()

        @plsc.parallel_loop(0, CHUNK, unroll=2)
        def row_body(r):
            acc = rows_u[slot, r, pl.ds(0, L)] * rows_v[slot, r, pl.ds(0, L)]
            for k in range(1, EMBD_DIM // L):
                acc = acc + (rows_u[slot, r, pl.ds(k * L, L)] *
                             rows_v[slot, r, pl.ds(k * L, L)])
            total = plsc.cumsum(acc)
            plsc.store_scatter(outb,
                               [jnp.full((L,), c * CHUNK + r, jnp.int32)],
                               total, mask=last_lane)

    pltpu.sync_copy(outb, out_hbm.at[pl.ds(base, B_PER_W)])


@jax.jit
def _ratings(user, item, w_user, w_item):
    mesh = plsc.VectorSubcoreMesh(core_axis_name="c", subcore_axis_name="s")
    return pl.kernel(
        _sc_body,
        out_type=jax.ShapeDtypeStruct((BATCH,), jnp.float32),
        mesh=mesh,
        compiler_params=pltpu.CompilerParams(needs_layout_passes=False),
        scratch_types=[
            pltpu.VMEM((B_PER_W,), jnp.int32),
            pltpu.VMEM((B_PER_W,), jnp.int32),
            pltpu.VMEM((NSLOT, CHUNK, EMBD_DIM), jnp.float32),
            pltpu.VMEM((NSLOT, CHUNK, EMBD_DIM), jnp.float32),
            pltpu.VMEM((B_PER_W,), jnp.float32),
            pltpu.SemaphoreType.DMA,
            pltpu.SemaphoreType.DMA,
            pltpu.SemaphoreType.DMA,
        ],
    )(user, item, w_user, w_item)


def kernel(user, item, W_user, W_item):
    return _ratings(user, item, W_user, W_item)


# speedup vs baseline: 1.0255x; 1.0124x over previous
import functools

import jax
import jax.numpy as jnp
from jax import lax
from jax.experimental import pallas as pl
from jax.experimental.pallas import tpu as pltpu
from jax.experimental.pallas import tpu_sc as plsc

BATCH = 16384
EMBD_DIM = 128
NC = 2
NS = 16
L = 16
NW = NC * NS
B_PER_W = BATCH // NW
CHUNK = 64
N_CHUNKS = B_PER_W // CHUNK
NSLOT = 3


def _sc_body(user_hbm, item_hbm, wu_hbm, wi_hbm, out_hbm,
             idx_u, idx_v, rows_u, rows_v, outb, sem0, sem1, sem2):
    wid = lax.axis_index("s") * NC + lax.axis_index("c")
    base = wid * B_PER_W
    last_lane = lax.iota(jnp.int32, L) == (L - 1)

    ci_u = pltpu.async_copy(user_hbm.at[pl.ds(base, B_PER_W)], idx_u, sem0)
    ci_v = pltpu.async_copy(item_hbm.at[pl.ds(base, B_PER_W)], idx_v, sem0)
    ci_u.wait()
    ci_v.wait()

    sems = [sem0, sem1, sem2]

    def start_gathers(c):
        slot = c % NSLOT
        cu = pltpu.async_copy(
            wu_hbm.at[idx_u.at[pl.ds(c * CHUNK, CHUNK)]],
            rows_u.at[slot], sems[slot])
        cv = pltpu.async_copy(
            wi_hbm.at[idx_v.at[pl.ds(c * CHUNK, CHUNK)]],
            rows_v.at[slot], sems[slot])
        return cu, cv

    pend = {0: start_gathers(0), 1: start_gathers(1)}
    for c in range(N_CHUNKS):
        cu, cv = pend[c % NSLOT]
        if c + 2 < N_CHUNKS:
            pend[(c + 2) % NSLOT] = start_gathers(c + 2)
        slot = c % NSLOT

        cu.wait()
        cv.wait()

        @plsc.parallel_loop(0, CHUNK, unroll=2)
        def row_body(r):
            acc = rows_u[slot, r, pl.ds(0, L)] * rows_v[slot, r, pl.ds(0, L)]
            for k in range(1, EMBD_DIM // L):
                acc = acc + (rows_u[slot, r, pl.ds(k * L, L)] *
                             rows_v[slot, r, pl.ds(k * L, L)])
            total = plsc.cumsum(acc)
            plsc.store_scatter(outb,
                               [jnp.full((L,), c * CHUNK + r, jnp.int32)],
                               total, mask=last_lane)

    pltpu.sync_copy(outb, out_hbm.at[pl.ds(base, B_PER_W)])


@jax.jit
def _ratings(user, item, w_user, w_item):
    mesh = plsc.VectorSubcoreMesh(core_axis_name="c", subcore_axis_name="s")
    return pl.kernel(
        _sc_body,
        out_type=jax.ShapeDtypeStruct((BATCH,), jnp.float32),
        mesh=mesh,
        compiler_params=pltpu.CompilerParams(needs_layout_passes=False),
        scratch_types=[
            pltpu.VMEM((B_PER_W,), jnp.int32),
            pltpu.VMEM((B_PER_W,), jnp.int32),
            pltpu.VMEM((NSLOT, CHUNK, EMBD_DIM), jnp.float32),
            pltpu.VMEM((NSLOT, CHUNK, EMBD_DIM), jnp.float32),
            pltpu.VMEM((B_PER_W,), jnp.float32),
            pltpu.SemaphoreType.DMA,
            pltpu.SemaphoreType.DMA,
            pltpu.SemaphoreType.DMA,
        ],
    )(user, item, w_user, w_item)


def kernel(user, item, W_user, W_item):
    return _ratings(user, item, W_user, W_item)


# speedup vs baseline: 1.0300x; 1.0045x over previous
import functools

import jax
import jax.numpy as jnp
from jax import lax
from jax.experimental import pallas as pl
from jax.experimental.pallas import tpu as pltpu
from jax.experimental.pallas import tpu_sc as plsc

BATCH = 16384
EMBD_DIM = 128
NC = 2
NS = 16
L = 16
NW = NC * NS
B_PER_W = BATCH // NW
CHUNK = 64
N_CHUNKS = B_PER_W // CHUNK
NSLOT = 4


def _sc_body(user_hbm, item_hbm, wu_hbm, wi_hbm, out_hbm,
             idx_u, idx_v, rows_u, rows_v, outb, sem0, sem1, sem2, sem3, sem4):
    wid = lax.axis_index("s") * NC + lax.axis_index("c")
    base = wid * B_PER_W
    last_lane = lax.iota(jnp.int32, L) == (L - 1)

    ci_u = pltpu.async_copy(user_hbm.at[pl.ds(base, B_PER_W)], idx_u, sem0)
    ci_v = pltpu.async_copy(item_hbm.at[pl.ds(base, B_PER_W)], idx_v, sem0)
    ci_u.wait()
    ci_v.wait()

    sems = [sem0, sem1, sem2, sem3]

    def start_gathers(c):
        slot = c % NSLOT
        cu = pltpu.async_copy(
            wu_hbm.at[idx_u.at[pl.ds(c * CHUNK, CHUNK)]],
            rows_u.at[slot], sems[slot])
        cv = pltpu.async_copy(
            wi_hbm.at[idx_v.at[pl.ds(c * CHUNK, CHUNK)]],
            rows_v.at[slot], sems[slot])
        return cu, cv

    pend = {0: start_gathers(0), 1: start_gathers(1), 2: start_gathers(2)}
    out_cps = []
    for c in range(N_CHUNKS):
        cu, cv = pend[c % NSLOT]
        if c + 3 < N_CHUNKS:
            pend[(c + 3) % NSLOT] = start_gathers(c + 3)
        slot = c % NSLOT

        cu.wait()
        cv.wait()

        @plsc.parallel_loop(0, CHUNK, unroll=2)
        def row_body(r):
            acc = rows_u[slot, r, pl.ds(0, L)] * rows_v[slot, r, pl.ds(0, L)]
            for k in range(1, EMBD_DIM // L):
                acc = acc + (rows_u[slot, r, pl.ds(k * L, L)] *
                             rows_v[slot, r, pl.ds(k * L, L)])
            total = plsc.cumsum(acc)
            plsc.store_scatter(outb,
                               [jnp.full((L,), c * CHUNK + r, jnp.int32)],
                               total, mask=last_lane)

        out_cps.append(pltpu.async_copy(
            outb.at[pl.ds(c * CHUNK, CHUNK)],
            out_hbm.at[pl.ds(base + c * CHUNK, CHUNK)], sem4))

    for cp in out_cps:
        cp.wait()


@jax.jit
def _ratings(user, item, w_user, w_item):
    mesh = plsc.VectorSubcoreMesh(core_axis_name="c", subcore_axis_name="s")
    return pl.kernel(
        _sc_body,
        out_type=jax.ShapeDtypeStruct((BATCH,), jnp.float32),
        mesh=mesh,
        compiler_params=pltpu.CompilerParams(needs_layout_passes=False),
        scratch_types=[
            pltpu.VMEM((B_PER_W,), jnp.int32),
            pltpu.VMEM((B_PER_W,), jnp.int32),
            pltpu.VMEM((NSLOT, CHUNK, EMBD_DIM), jnp.float32),
            pltpu.VMEM((NSLOT, CHUNK, EMBD_DIM), jnp.float32),
            pltpu.VMEM((B_PER_W,), jnp.float32),
            pltpu.SemaphoreType.DMA,
            pltpu.SemaphoreType.DMA,
            pltpu.SemaphoreType.DMA,
            pltpu.SemaphoreType.DMA,
            pltpu.SemaphoreType.DMA,
        ],
    )(user, item, w_user, w_item)


def kernel(user, item, W_user, W_item):
    return _ratings(user, item, W_user, W_item)
